# baseline (speedup 1.0000x reference)
import jax
import jax.numpy as jnp
from jax import lax
from jax.experimental import pallas as pl
from jax.experimental.pallas import tpu as pltpu
from jax.experimental.pallas import tpu_sc as plsc

VOCAB = 1000000
EMBED_DIM = 32
BATCH = 16384

_info = plsc.get_sparse_core_info()
_NC = _info.num_cores
_NS = _info.num_subcores
_NW = _NC * _NS
_CHUNK = 128
_NCHUNK = BATCH // _CHUNK
_WIN = 128
_NBUF = 4


def _gather_body(idx_hbm, table_t_hbm, out_t_hbm, idx_v, buf_v, outc_v, *sems):
    wid = lax.axis_index("s") * _NC + lax.axis_index("c")
    col_ref = table_t_hbm.at[wid]
    out_col = out_t_hbm.at[wid]
    pltpu.sync_copy(idx_hbm, idx_v)
    lane_win = lax.iota(jnp.int32, 16) * _WIN
    half = _CHUNK * _WIN

    def fire(k, b):
        kbase = k * _CHUNK

        def body(g, carry):
            vec = idx_v[pl.ds(kbase + g * 16, 16)]
            basev = vec & ~(_WIN - 1)
            for l in range(16):
                pltpu.make_async_copy(
                    col_ref.at[pl.ds(pl.multiple_of(basev[l], _WIN), _WIN)],
                    buf_v.at[
                        pl.ds(
                            pl.multiple_of(b * half + (g * 16 + l) * _WIN, _WIN),
                            _WIN,
                        )
                    ],
                    sems[b],
                ).start()
            return carry

        lax.fori_loop(0, _CHUNK // 16, body, 0)

    def drain(b):
        pltpu.make_async_copy(
            col_ref.at[pl.ds(0, half)],
            buf_v.at[pl.ds(b * half, half)],
            sems[b],
        ).wait()

    def select_out(k, b):
        kbase = k * _CHUNK

        def body(g, carry):
            vec = idx_v[pl.ds(kbase + g * 16, 16)]
            flat = b * half + g * (16 * _WIN) + lane_win + (vec & (_WIN - 1))
            outc_v[pl.ds(g * 16, 16)] = plsc.load_gather(buf_v, [flat])
            return carry

        lax.fori_loop(0, _CHUNK // 16, body, 0)
        pltpu.sync_copy(
            outc_v, out_col.at[pl.ds(pl.multiple_of(kbase, _CHUNK), _CHUNK)]
        )

    ngroup = _NCHUNK // _NBUF
    for b in range(_NBUF - 1):
        fire(b, b)

    def group(m, carry):
        for j in range(_NBUF):
            k = m * _NBUF + j

            @pl.when(k + _NBUF - 1 < _NCHUNK)
            def _():
                fire(k + _NBUF - 1, (j + _NBUF - 1) % _NBUF)

            drain(j)
            select_out(k, j)
        return carry

    lax.fori_loop(0, ngroup, group, 0)


@jax.jit
def _sc_gather(idx, table_t):
    mesh = plsc.VectorSubcoreMesh(core_axis_name="c", subcore_axis_name="s")
    run = pl.kernel(
        _gather_body,
        mesh=mesh,
        out_type=jax.ShapeDtypeStruct((EMBED_DIM, BATCH), jnp.float32),
        scratch_types=[
            pltpu.VMEM((BATCH,), jnp.int32),
            pltpu.VMEM((_NBUF * _CHUNK * _WIN,), jnp.float32),
            pltpu.VMEM((_CHUNK,), jnp.float32),
        ] + [pltpu.SemaphoreType.DMA] * _NBUF,
        compiler_params=pltpu.CompilerParams(
            use_tc_tiling_on_sc=True, needs_layout_passes=False
        ),
    )
    return run(idx, table_t)


def kernel(customer_id, emb_table):
    idx = customer_id.astype(jnp.int32)
    out_t = _sc_gather(idx, emb_table.T)
    return out_t.T
